# Initial kernel scaffold; baseline (speedup 1.0000x reference)
import functools

import jax
import jax.numpy as jnp
from jax import lax
from jax.experimental import pallas as pl
from jax.experimental.pallas import tpu as pltpu
from jax.experimental.pallas import tpu_sc as plsc

N = 10000
D = 128
NCLS = 64
E = 320000

NC = 2
NS = 16
NW = NC * NS

CHUNK = 128
CHUNKS = 79
EPT = CHUNKS * CHUNK
EPAD = NW * EPT
DUMP = N
ACC_ROWS = 10240
ZROWS = ACC_ROWS // NS
OROWS = N // NS

_MESH = plsc.VectorSubcoreMesh(core_axis_name="c", subcore_axis_name="s")


@functools.partial(
    pl.kernel,
    out_type=jax.ShapeDtypeStruct((2 * N, 8), jnp.float32),
    mesh=_MESH,
    scratch_types=[
        pltpu.VMEM_SHARED((ACC_ROWS, 8), jnp.float32),
        pltpu.VMEM((CHUNK,), jnp.int32),
        pltpu.VMEM((CHUNK, 8), jnp.float32),
    ],
)
def _sc_degree(dst_hbm, zeros_hbm, ones_hbm, out_hbm, acc, dst_v, ones_v):
    c = lax.axis_index("c")
    s = lax.axis_index("s")
    w = c * NS + s
    pltpu.sync_copy(zeros_hbm, acc.at[pl.ds(s * ZROWS, ZROWS)])
    pltpu.sync_copy(ones_hbm, ones_v)
    plsc.subcore_barrier()

    base = w * EPT

    def body(j, carry):
        off = pl.multiple_of(base + j * CHUNK, CHUNK)
        pltpu.sync_copy(dst_hbm.at[pl.ds(off, CHUNK)], dst_v)
        pltpu.sync_copy(ones_v, acc.at[dst_v], add=True)
        return carry

    lax.fori_loop(0, CHUNKS, body, 0)
    plsc.subcore_barrier()
    pltpu.sync_copy(
        acc.at[pl.ds(s * OROWS, OROWS)],
        out_hbm.at[pl.ds(c * N + s * OROWS, OROWS)],
    )


@functools.partial(
    pl.kernel,
    out_type=jax.ShapeDtypeStruct((2 * N, D), jnp.float32),
    mesh=_MESH,
    scratch_types=[
        pltpu.VMEM_SHARED((ACC_ROWS, D), jnp.float32),
        pltpu.VMEM((CHUNK,), jnp.int32),
        pltpu.VMEM((CHUNK,), jnp.int32),
        pltpu.VMEM((CHUNK, D), jnp.float32),
        pltpu.SemaphoreType.DMA,
    ],
)
def _sc_prop(hs_hbm, src_hbm, dst_hbm, zeros_hbm, out_hbm,
             acc, src_v, dst_v, rows_v, sem):
    c = lax.axis_index("c")
    s = lax.axis_index("s")
    w = c * NS + s
    pltpu.sync_copy(zeros_hbm, acc.at[pl.ds(s * ZROWS, ZROWS)])
    plsc.subcore_barrier()

    base = w * EPT

    def body(j, carry):
        off = pl.multiple_of(base + j * CHUNK, CHUNK)
        pltpu.sync_copy(src_hbm.at[pl.ds(off, CHUNK)], src_v)
        pltpu.async_copy(hs_hbm.at[src_v], rows_v, sem).wait()
        pltpu.sync_copy(dst_hbm.at[pl.ds(off, CHUNK)], dst_v)
        pltpu.sync_copy(rows_v, acc.at[dst_v], add=True)
        return carry

    lax.fori_loop(0, CHUNKS, body, 0)
    plsc.subcore_barrier()
    pltpu.sync_copy(
        acc.at[pl.ds(s * OROWS, OROWS)],
        out_hbm.at[pl.ds(c * N + s * OROWS, OROWS)],
    )


_R = 1000


def _tc_pre_body(deg0, deg1, x, w1, dis, h, hs):
    d = lax.rsqrt(deg0[:, 0:1] + deg1[:, 0:1] + 1.0)
    hh = jnp.dot(x[...], w1[...], preferred_element_type=jnp.float32)
    dis[...] = d
    h[...] = hh
    hs[...] = d * hh


def _tc_pre(deg, x, W1):
    grid = (N // _R,)
    return pl.pallas_call(
        _tc_pre_body,
        grid=grid,
        in_specs=[
            pl.BlockSpec((_R, 8), lambda i: (i, 0)),
            pl.BlockSpec((_R, 8), lambda i: (i + N // _R, 0)),
            pl.BlockSpec((_R, D), lambda i: (i, 0)),
            pl.BlockSpec((D, D), lambda i: (0, 0)),
        ],
        out_specs=[
            pl.BlockSpec((_R, 1), lambda i: (i, 0)),
            pl.BlockSpec((_R, D), lambda i: (i, 0)),
            pl.BlockSpec((_R, D), lambda i: (i, 0)),
        ],
        out_shape=[
            jax.ShapeDtypeStruct((N, 1), jnp.float32),
            jax.ShapeDtypeStruct((N, D), jnp.float32),
            jax.ShapeDtypeStruct((N, D), jnp.float32),
        ],
    )(deg, deg, x, W1)


def _tc_mid_body(acc0, acc1, h1, dis, b1, w2, h2, hs2):
    d = dis[...]
    u = d * (acc0[...] + acc1[...]) + (d * d) * h1[...] + b1[...]
    u = jnp.maximum(u, 0.0)
    hh = jnp.dot(u, w2[...], preferred_element_type=jnp.float32)
    h2[...] = hh
    hs2[...] = d * hh


def _tc_mid(acc, h1, dis, b1, W2):
    grid = (N // _R,)
    return pl.pallas_call(
        _tc_mid_body,
        grid=grid,
        in_specs=[
            pl.BlockSpec((_R, D), lambda i: (i, 0)),
            pl.BlockSpec((_R, D), lambda i: (i + N // _R, 0)),
            pl.BlockSpec((_R, D), lambda i: (i, 0)),
            pl.BlockSpec((_R, 1), lambda i: (i, 0)),
            pl.BlockSpec((1, D), lambda i: (0, 0)),
            pl.BlockSpec((D, D), lambda i: (0, 0)),
        ],
        out_specs=[
            pl.BlockSpec((_R, D), lambda i: (i, 0)),
            pl.BlockSpec((_R, D), lambda i: (i, 0)),
        ],
        out_shape=[
            jax.ShapeDtypeStruct((N, D), jnp.float32),
            jax.ShapeDtypeStruct((N, D), jnp.float32),
        ],
    )(acc, acc, h1, dis, b1, W2)


def _tc_fin_body(acc0, acc1, h2, dis, b2, wfc, bfc, out):
    d = dis[...]
    u = d * (acc0[...] + acc1[...]) + (d * d) * h2[...] + b2[...]
    u = jnp.maximum(u, 0.0)
    logits = jnp.dot(u, wfc[...], preferred_element_type=jnp.float32)
    logits = logits + bfc[...]
    m = jnp.max(logits, axis=1, keepdims=True)
    e = jnp.exp(logits - m)
    out[...] = e / jnp.sum(e, axis=1, keepdims=True)


def _tc_fin(acc, h2, dis, b2, Wfc, bfc):
    grid = (N // _R,)
    return pl.pallas_call(
        _tc_fin_body,
        grid=grid,
        in_specs=[
            pl.BlockSpec((_R, D), lambda i: (i, 0)),
            pl.BlockSpec((_R, D), lambda i: (i + N // _R, 0)),
            pl.BlockSpec((_R, D), lambda i: (i, 0)),
            pl.BlockSpec((_R, 1), lambda i: (i, 0)),
            pl.BlockSpec((1, D), lambda i: (0, 0)),
            pl.BlockSpec((D, NCLS), lambda i: (0, 0)),
            pl.BlockSpec((1, NCLS), lambda i: (0, 0)),
        ],
        out_specs=pl.BlockSpec((_R, NCLS), lambda i: (i, 0)),
        out_shape=jax.ShapeDtypeStruct((N, NCLS), jnp.float32),
    )(acc, acc, h2, dis, b2, Wfc, bfc)


def kernel(x, edge_index, W1, b1, W2, b2, Wfc, bfc):
    src = edge_index[0].astype(jnp.int32)
    dst = edge_index[1].astype(jnp.int32)
    pad = EPAD - E
    srcp = jnp.concatenate([src, jnp.zeros((pad,), jnp.int32)])
    dstp = jnp.concatenate([dst, jnp.full((pad,), DUMP, jnp.int32)])

    zeros_d = jnp.zeros((ZROWS, D), jnp.float32)
    zeros_8 = jnp.zeros((ZROWS, 8), jnp.float32)
    ones_8 = jnp.ones((CHUNK, 8), jnp.float32)

    deg = _sc_degree(dstp, zeros_8, ones_8)
    dis, h1, hs1 = _tc_pre(deg, x, W1)
    acc1 = _sc_prop(hs1, srcp, dstp, zeros_d)
    h2, hs2 = _tc_mid(acc1, h1, dis, b1.reshape(1, D), W2)
    acc2 = _sc_prop(hs2, srcp, dstp, zeros_d)
    out = _tc_fin(acc2, h2, dis, b2.reshape(1, D), Wfc, bfc.reshape(1, NCLS))
    return out


# speedup vs baseline: 10.0850x; 10.0850x over previous
import functools

import jax
import jax.numpy as jnp
from jax import lax
from jax.experimental import pallas as pl
from jax.experimental.pallas import tpu as pltpu
from jax.experimental.pallas import tpu_sc as plsc

N = 10000
D = 128
NCLS = 64
E = 320000

NC = 2
NS = 16
NW = NC * NS

CHUNK = 128
CHUNKS = 79
EPT = CHUNKS * CHUNK
EPAD = NW * EPT
DUMP = N
ACC_ROWS = 10240
ZROWS = ACC_ROWS // NS
OROWS = N // NS

_MESH = plsc.VectorSubcoreMesh(core_axis_name="c", subcore_axis_name="s")


@functools.partial(
    pl.kernel,
    out_type=jax.ShapeDtypeStruct((NC, ACC_ROWS, D), jnp.float32),
    mesh=_MESH,
    scratch_types=[
        pltpu.VMEM_SHARED((ACC_ROWS, D), jnp.float32),
        pltpu.VMEM((CHUNK,), jnp.int32),
        pltpu.VMEM((CHUNK, D), jnp.float32),
    ],
)
def _sc_degree(dst_hbm, zeros_hbm, ones_hbm, out_hbm, acc, dst_v, ones_v):
    c = lax.axis_index("c")
    s = lax.axis_index("s")
    w = c * NS + s
    pltpu.sync_copy(zeros_hbm, acc.at[pl.ds(s * ZROWS, ZROWS)])
    pltpu.sync_copy(ones_hbm, ones_v)
    plsc.subcore_barrier()

    base = w * EPT

    def body(j, carry):
        off = pl.multiple_of(base + j * CHUNK, CHUNK)
        pltpu.sync_copy(dst_hbm.at[pl.ds(off, CHUNK)], dst_v)
        pltpu.sync_copy(ones_v, acc.at[dst_v], add=True)
        return carry

    lax.fori_loop(0, CHUNKS, body, 0)
    plsc.subcore_barrier()
    pltpu.sync_copy(
        acc.at[pl.ds(s * ZROWS, ZROWS)],
        out_hbm.at[c, pl.ds(s * ZROWS, ZROWS)],
    )


@functools.partial(
    pl.kernel,
    out_type=jax.ShapeDtypeStruct((NC, ACC_ROWS, D), jnp.float32),
    mesh=_MESH,
    scratch_types=[
        pltpu.VMEM_SHARED((ACC_ROWS, D), jnp.float32),
        pltpu.VMEM((CHUNK,), jnp.int32),
        pltpu.VMEM((CHUNK,), jnp.int32),
        pltpu.VMEM((CHUNK, D), jnp.float32),
        pltpu.SemaphoreType.DMA,
    ],
)
def _sc_prop(hs_hbm, src_hbm, dst_hbm, zeros_hbm, out_hbm,
             acc, src_v, dst_v, rows_v, sem):
    c = lax.axis_index("c")
    s = lax.axis_index("s")
    w = c * NS + s
    pltpu.sync_copy(zeros_hbm, acc.at[pl.ds(s * ZROWS, ZROWS)])
    plsc.subcore_barrier()

    base = w * EPT

    def body(j, carry):
        off = pl.multiple_of(base + j * CHUNK, CHUNK)
        pltpu.sync_copy(src_hbm.at[pl.ds(off, CHUNK)], src_v)
        pltpu.async_copy(hs_hbm.at[src_v], rows_v, sem).wait()
        pltpu.sync_copy(dst_hbm.at[pl.ds(off, CHUNK)], dst_v)
        pltpu.sync_copy(rows_v, acc.at[dst_v], add=True)
        return carry

    lax.fori_loop(0, CHUNKS, body, 0)
    plsc.subcore_barrier()
    pltpu.sync_copy(
        acc.at[pl.ds(s * ZROWS, ZROWS)],
        out_hbm.at[c, pl.ds(s * ZROWS, ZROWS)],
    )


_R = 1000


def _tc_pre_body(deg0, deg1, x, w1, dis, h, hs):
    d = lax.rsqrt(deg0[0, :, 0:1] + deg1[0, :, 0:1] + 1.0)
    hh = jnp.dot(x[...], w1[...], preferred_element_type=jnp.float32)
    dis[...] = d
    h[...] = hh
    hs[...] = d * hh


def _tc_pre(deg, x, W1):
    grid = (N // _R,)
    return pl.pallas_call(
        _tc_pre_body,
        grid=grid,
        in_specs=[
            pl.BlockSpec((1, _R, D), lambda i: (0, i, 0)),
            pl.BlockSpec((1, _R, D), lambda i: (1, i, 0)),
            pl.BlockSpec((_R, D), lambda i: (i, 0)),
            pl.BlockSpec((D, D), lambda i: (0, 0)),
        ],
        out_specs=[
            pl.BlockSpec((_R, 1), lambda i: (i, 0)),
            pl.BlockSpec((_R, D), lambda i: (i, 0)),
            pl.BlockSpec((_R, D), lambda i: (i, 0)),
        ],
        out_shape=[
            jax.ShapeDtypeStruct((N, 1), jnp.float32),
            jax.ShapeDtypeStruct((N, D), jnp.float32),
            jax.ShapeDtypeStruct((N, D), jnp.float32),
        ],
    )(deg, deg, x, W1)


def _tc_mid_body(acc0, acc1, h1, dis, b1, w2, h2, hs2):
    d = dis[...]
    u = d * (acc0[0] + acc1[0]) + (d * d) * h1[...] + b1[...]
    u = jnp.maximum(u, 0.0)
    hh = jnp.dot(u, w2[...], preferred_element_type=jnp.float32)
    h2[...] = hh
    hs2[...] = d * hh


def _tc_mid(acc, h1, dis, b1, W2):
    grid = (N // _R,)
    return pl.pallas_call(
        _tc_mid_body,
        grid=grid,
        in_specs=[
            pl.BlockSpec((1, _R, D), lambda i: (0, i, 0)),
            pl.BlockSpec((1, _R, D), lambda i: (1, i, 0)),
            pl.BlockSpec((_R, D), lambda i: (i, 0)),
            pl.BlockSpec((_R, 1), lambda i: (i, 0)),
            pl.BlockSpec((1, D), lambda i: (0, 0)),
            pl.BlockSpec((D, D), lambda i: (0, 0)),
        ],
        out_specs=[
            pl.BlockSpec((_R, D), lambda i: (i, 0)),
            pl.BlockSpec((_R, D), lambda i: (i, 0)),
        ],
        out_shape=[
            jax.ShapeDtypeStruct((N, D), jnp.float32),
            jax.ShapeDtypeStruct((N, D), jnp.float32),
        ],
    )(acc, acc, h1, dis, b1, W2)


def _tc_fin_body(acc0, acc1, h2, dis, b2, wfc, bfc, out):
    d = dis[...]
    u = d * (acc0[0] + acc1[0]) + (d * d) * h2[...] + b2[...]
    u = jnp.maximum(u, 0.0)
    logits = jnp.dot(u, wfc[...], preferred_element_type=jnp.float32)
    logits = logits + bfc[...]
    m = jnp.max(logits, axis=1, keepdims=True)
    e = jnp.exp(logits - m)
    out[...] = e / jnp.sum(e, axis=1, keepdims=True)


def _tc_fin(acc, h2, dis, b2, Wfc, bfc):
    grid = (N // _R,)
    return pl.pallas_call(
        _tc_fin_body,
        grid=grid,
        in_specs=[
            pl.BlockSpec((1, _R, D), lambda i: (0, i, 0)),
            pl.BlockSpec((1, _R, D), lambda i: (1, i, 0)),
            pl.BlockSpec((_R, D), lambda i: (i, 0)),
            pl.BlockSpec((_R, 1), lambda i: (i, 0)),
            pl.BlockSpec((1, D), lambda i: (0, 0)),
            pl.BlockSpec((D, NCLS), lambda i: (0, 0)),
            pl.BlockSpec((1, NCLS), lambda i: (0, 0)),
        ],
        out_specs=pl.BlockSpec((_R, NCLS), lambda i: (i, 0)),
        out_shape=jax.ShapeDtypeStruct((N, NCLS), jnp.float32),
    )(acc, acc, h2, dis, b2, Wfc, bfc)


def kernel(x, edge_index, W1, b1, W2, b2, Wfc, bfc):
    src = edge_index[0].astype(jnp.int32)
    dst = edge_index[1].astype(jnp.int32)
    pad = EPAD - E
    srcp = jnp.concatenate([src, jnp.zeros((pad,), jnp.int32)])
    dstp = jnp.concatenate([dst, jnp.full((pad,), DUMP, jnp.int32)])

    zeros_d = jnp.zeros((ZROWS, D), jnp.float32)
    ones_d = jnp.ones((CHUNK, D), jnp.float32)

    deg = _sc_degree(dstp, zeros_d, ones_d)
    dis, h1, hs1 = _tc_pre(deg, x, W1)
    acc1 = _sc_prop(hs1, srcp, dstp, zeros_d)
    h2, hs2 = _tc_mid(acc1, h1, dis, b1.reshape(1, D), W2)
    acc2 = _sc_prop(hs2, srcp, dstp, zeros_d)
    out = _tc_fin(acc2, h2, dis, b2.reshape(1, D), Wfc, bfc.reshape(1, NCLS))
    return out
